# Initial kernel scaffold; baseline (speedup 1.0000x reference)
import jax
import jax.numpy as jnp
from jax.experimental import pallas as pl


def kernel(x, edge_index, W, b):
    raise NotImplementedError("write your pallas kernel here")


# speedup vs baseline: 14.1839x; 14.1839x over previous
import jax
import jax.numpy as jnp
from jax import lax
from jax.experimental import pallas as pl
from jax.experimental.pallas import tpu as pltpu
from jax.experimental.pallas import tpu_sc as plsc

_LANES = 128
_NSC = 2
_NSUB = 16
_NW = _NSC * _NSUB


def _cdiv(a, b):
    return (a + b - 1) // b


def _sc_mesh():
    return plsc.VectorSubcoreMesh(core_axis_name="c", subcore_axis_name="s")


def _sc_degree(dst2d, zeros16, ones16, n_pad, nch_w):
    rows_w = n_pad // _NSUB

    def body(dst_hbm, zeros_hbm, ones_hbm, out_hbm, didx_all, ones_v, acc):
        cid = lax.axis_index("c")
        sid = lax.axis_index("s")
        wid = cid * _NSUB + sid
        pltpu.sync_copy(zeros_hbm, acc.at[pl.ds(sid * rows_w, rows_w)])
        pltpu.sync_copy(ones_hbm, ones_v)
        pltpu.sync_copy(dst_hbm.at[pl.ds(wid * nch_w, nch_w)], didx_all)
        plsc.subcore_barrier()

        def step(ci, _):
            pltpu.sync_copy(ones_v, acc.at[didx_all.at[ci]], add=True)
            return ()

        lax.fori_loop(0, nch_w, step, ())
        plsc.subcore_barrier()
        pltpu.sync_copy(acc.at[pl.ds(sid * rows_w, rows_w)],
                        out_hbm.at[cid, pl.ds(sid * rows_w, rows_w)])

    fn = pl.kernel(
        body,
        out_type=jax.ShapeDtypeStruct((_NSC, n_pad, 16), jnp.float32),
        mesh=_sc_mesh(),
        compiler_params=pltpu.CompilerParams(use_tc_tiling_on_sc=False),
        scratch_types=[
            pltpu.VMEM((nch_w, _LANES), jnp.int32),
            pltpu.VMEM((_LANES, 16), jnp.float32),
            pltpu.VMEM_SHARED((n_pad, 16), jnp.float32),
        ],
    )
    return fn(dst2d, zeros16, ones16)


def _sc_spmm(y, src2d, dst2d, zeros_f, n_pad, nch_w):
    f = y.shape[1]
    rows_w = n_pad // _NSUB

    def body(y_hbm, src_hbm, dst_hbm, zeros_hbm, out_hbm,
             sidx_all, didx_all, rows, acc, sem):
        cid = lax.axis_index("c")
        sid = lax.axis_index("s")
        wid = cid * _NSUB + sid
        pltpu.sync_copy(zeros_hbm, acc.at[pl.ds(sid * rows_w, rows_w)])
        pltpu.sync_copy(src_hbm.at[pl.ds(wid * nch_w, nch_w)], sidx_all)
        pltpu.sync_copy(dst_hbm.at[pl.ds(wid * nch_w, nch_w)], didx_all)
        plsc.subcore_barrier()

        def step(ci, _):
            pltpu.async_copy(y_hbm.at[sidx_all.at[ci]], rows, sem).wait()
            pltpu.sync_copy(rows, acc.at[didx_all.at[ci]], add=True)
            return ()

        lax.fori_loop(0, nch_w, step, ())
        plsc.subcore_barrier()
        pltpu.sync_copy(acc.at[pl.ds(sid * rows_w, rows_w)],
                        out_hbm.at[cid, pl.ds(sid * rows_w, rows_w)])

    fn = pl.kernel(
        body,
        out_type=jax.ShapeDtypeStruct((_NSC, n_pad, f), jnp.float32),
        mesh=_sc_mesh(),
        compiler_params=pltpu.CompilerParams(use_tc_tiling_on_sc=False),
        scratch_types=[
            pltpu.VMEM((nch_w, _LANES), jnp.int32),
            pltpu.VMEM((nch_w, _LANES), jnp.int32),
            pltpu.VMEM((_LANES, f), jnp.float32),
            pltpu.VMEM_SHARED((n_pad, f), jnp.float32),
            pltpu.SemaphoreType.DMA,
        ],
    )
    return fn(y, src2d, dst2d, zeros_f)


def _deg_from_partials(degp_ref, n):
    deg = (degp_ref[0] + degp_ref[1]).sum(axis=-1) * (1.0 / 16.0) + 1.0
    return deg[:n]


def _tc_matmul(x, W):
    def body(x_ref, w_ref, o_ref):
        o_ref[...] = jnp.dot(x_ref[...], w_ref[...],
                             preferred_element_type=jnp.float32)

    return pl.pallas_call(
        body,
        out_shape=jax.ShapeDtypeStruct((x.shape[0], W.shape[1]), jnp.float32),
    )(x, W)


def _tc_scale_first(degp, xw, n):
    def body(degp_ref, xw_ref, z_ref):
        dis = lax.rsqrt(_deg_from_partials(degp_ref, n))
        z_ref[...] = xw_ref[...] * dis[:, None]

    return pl.pallas_call(
        body,
        out_shape=jax.ShapeDtypeStruct(xw.shape, jnp.float32),
    )(degp, xw)


def _tc_mid(degp, up, z, n):
    def body(degp_ref, up_ref, z_ref, v_ref):
        dinv = 1.0 / _deg_from_partials(degp_ref, n)
        s = up_ref[0, :n, :] + up_ref[1, :n, :] + z_ref[...]
        v_ref[...] = s * dinv[:, None]

    return pl.pallas_call(
        body,
        out_shape=jax.ShapeDtypeStruct(z.shape, jnp.float32),
    )(degp, up, z)


def _tc_final(degp, wp, v, b2d, n):
    def body(degp_ref, wp_ref, v_ref, b_ref, o_ref):
        dis = lax.rsqrt(_deg_from_partials(degp_ref, n))
        logits = (wp_ref[0, :n, :] + wp_ref[1, :n, :] + v_ref[...])
        logits = logits * dis[:, None] + b_ref[...]
        m = jnp.max(logits, axis=-1, keepdims=True)
        ex = jnp.exp(logits - m)
        lse = jnp.log(jnp.sum(ex, axis=-1, keepdims=True)) + m
        o_ref[...] = logits - lse

    return pl.pallas_call(
        body,
        out_shape=jax.ShapeDtypeStruct(v.shape, jnp.float32),
    )(degp, wp, v, b2d)


def kernel(x, edge_index, W, b):
    n = x.shape[0]
    c_out = W.shape[1]
    e = edge_index.shape[1]

    n_pad = _cdiv(n + 1, _LANES) * _LANES
    nch = _cdiv(e, _LANES * _NW * 8) * _NW * 8
    nch_w = nch // _NW
    ep = nch * _LANES

    src = edge_index[0]
    dst = edge_index[1]
    pad = ep - e
    src_p = jnp.concatenate(
        [src, jnp.zeros((pad,), src.dtype)]).reshape(nch, _LANES)
    dst_p = jnp.concatenate(
        [dst, jnp.full((pad,), n, dst.dtype)]).reshape(nch, _LANES)

    rows_w = n_pad // _NSUB
    zeros16 = jnp.zeros((rows_w, 16), jnp.float32)
    zerosf = jnp.zeros((rows_w, c_out), jnp.float32)
    ones16 = jnp.ones((_LANES, 16), jnp.float32)

    xw = _tc_matmul(x, W)
    degp = _sc_degree(dst_p, zeros16, ones16, n_pad, nch_w)
    z = _tc_scale_first(degp, xw, n)
    up = _sc_spmm(z, src_p, dst_p, zerosf, n_pad, nch_w)
    v = _tc_mid(degp, up, z, n)
    wp = _sc_spmm(v, src_p, dst_p, zerosf, n_pad, nch_w)
    return _tc_final(degp, wp, v, b.reshape(1, -1), n)


# speedup vs baseline: 37.4710x; 2.6418x over previous
import jax
import jax.numpy as jnp
from jax import lax
from jax.experimental import pallas as pl
from jax.experimental.pallas import tpu as pltpu
from jax.experimental.pallas import tpu_sc as plsc

_LANES = 128
_NSC = 2
_NSUB = 16
_NW = _NSC * _NSUB


def _cdiv(a, b):
    return (a + b - 1) // b


def _sc_mesh():
    return plsc.VectorSubcoreMesh(core_axis_name="c", subcore_axis_name="s")


def _sc_degree(dst2d, zeros16, ones16, n_pad, nch_w):
    rows_w = n_pad // _NSUB

    def body(dst_hbm, zeros_hbm, ones_hbm, out_hbm, didx_all, ones_v, acc):
        cid = lax.axis_index("c")
        sid = lax.axis_index("s")
        wid = cid * _NSUB + sid
        pltpu.sync_copy(zeros_hbm, acc.at[pl.ds(sid * rows_w, rows_w)])
        pltpu.sync_copy(ones_hbm, ones_v)
        pltpu.sync_copy(dst_hbm.at[pl.ds(wid * nch_w, nch_w)], didx_all)
        plsc.subcore_barrier()

        def step(ci, _):
            pltpu.sync_copy(ones_v, acc.at[didx_all.at[ci]], add=True)
            return ()

        lax.fori_loop(0, nch_w, step, ())
        plsc.subcore_barrier()
        pltpu.sync_copy(acc.at[pl.ds(sid * rows_w, rows_w)],
                        out_hbm.at[cid, pl.ds(sid * rows_w, rows_w)])

    fn = pl.kernel(
        body,
        out_type=jax.ShapeDtypeStruct((_NSC, n_pad, 16), jnp.float32),
        mesh=_sc_mesh(),
        compiler_params=pltpu.CompilerParams(use_tc_tiling_on_sc=False),
        scratch_types=[
            pltpu.VMEM((nch_w, _LANES), jnp.int32),
            pltpu.VMEM((_LANES, 16), jnp.float32),
            pltpu.VMEM_SHARED((n_pad, 16), jnp.float32),
        ],
    )
    return fn(dst2d, zeros16, ones16)


def _sc_spmm(y, src2d, dst2d, zeros_f, n_pad, nch_w):
    f = y.shape[1]
    rows_w = n_pad // _NSUB

    npairs = nch_w // 2

    def body(y_hbm, src_hbm, dst_hbm, zeros_hbm, out_hbm,
             sidx_all, didx_all, rows0, rows1, acc, gsem0, gsem1):
        cid = lax.axis_index("c")
        sid = lax.axis_index("s")
        wid = cid * _NSUB + sid
        pltpu.sync_copy(zeros_hbm, acc.at[pl.ds(sid * rows_w, rows_w)])
        pltpu.sync_copy(src_hbm.at[pl.ds(wid * nch_w, nch_w)], sidx_all)
        pltpu.sync_copy(dst_hbm.at[pl.ds(wid * nch_w, nch_w)], didx_all)
        plsc.subcore_barrier()

        pltpu.async_copy(y_hbm.at[sidx_all.at[0]], rows0, gsem0)

        def step(i, _):
            c0 = 2 * i
            c1 = c0 + 1
            pltpu.async_copy(y_hbm.at[sidx_all.at[c1]], rows1, gsem1)
            pltpu.make_async_copy(y_hbm.at[sidx_all.at[c0]], rows0, gsem0).wait()
            pltpu.sync_copy(rows0, acc.at[didx_all.at[c0]], add=True)
            cn = jnp.minimum(c0 + 2, nch_w - 1)
            pltpu.async_copy(y_hbm.at[sidx_all.at[cn]], rows0, gsem0)
            pltpu.make_async_copy(y_hbm.at[sidx_all.at[c1]], rows1, gsem1).wait()
            pltpu.sync_copy(rows1, acc.at[didx_all.at[c1]], add=True)
            return ()

        lax.fori_loop(0, npairs, step, ())
        pltpu.make_async_copy(y_hbm.at[sidx_all.at[nch_w - 1]], rows0,
                              gsem0).wait()
        plsc.subcore_barrier()
        pltpu.sync_copy(acc.at[pl.ds(sid * rows_w, rows_w)],
                        out_hbm.at[cid, pl.ds(sid * rows_w, rows_w)])

    fn = pl.kernel(
        body,
        out_type=jax.ShapeDtypeStruct((_NSC, n_pad, f), jnp.float32),
        mesh=_sc_mesh(),
        compiler_params=pltpu.CompilerParams(use_tc_tiling_on_sc=False),
        scratch_types=[
            pltpu.VMEM((nch_w, _LANES), jnp.int32),
            pltpu.VMEM((nch_w, _LANES), jnp.int32),
            pltpu.VMEM((_LANES, f), jnp.float32),
            pltpu.VMEM((_LANES, f), jnp.float32),
            pltpu.VMEM_SHARED((n_pad, f), jnp.float32),
            pltpu.SemaphoreType.DMA,
            pltpu.SemaphoreType.DMA,
        ],
    )
    return fn(y, src2d, dst2d, zeros_f)


def _deg_from_partials(degp_ref, n):
    deg = (degp_ref[0] + degp_ref[1]).sum(axis=-1) * (1.0 / 16.0) + 1.0
    return deg[:n]


def _tc_matmul(x, W):
    def body(x_ref, w_ref, o_ref):
        o_ref[...] = jnp.dot(x_ref[...], w_ref[...],
                             preferred_element_type=jnp.float32)

    return pl.pallas_call(
        body,
        out_shape=jax.ShapeDtypeStruct((x.shape[0], W.shape[1]), jnp.float32),
    )(x, W)


def _tc_scale_first(degp, xw, n):
    def body(degp_ref, xw_ref, z_ref):
        dis = lax.rsqrt(_deg_from_partials(degp_ref, n))
        z_ref[...] = xw_ref[...] * dis[:, None]

    return pl.pallas_call(
        body,
        out_shape=jax.ShapeDtypeStruct(xw.shape, jnp.float32),
    )(degp, xw)


def _tc_mid(degp, up, z, n):
    def body(degp_ref, up_ref, z_ref, v_ref):
        dinv = 1.0 / _deg_from_partials(degp_ref, n)
        s = up_ref[0, :n, :] + up_ref[1, :n, :] + z_ref[...]
        v_ref[...] = s * dinv[:, None]

    return pl.pallas_call(
        body,
        out_shape=jax.ShapeDtypeStruct(z.shape, jnp.float32),
    )(degp, up, z)


def _tc_final(degp, wp, v, b2d, n):
    def body(degp_ref, wp_ref, v_ref, b_ref, o_ref):
        dis = lax.rsqrt(_deg_from_partials(degp_ref, n))
        logits = (wp_ref[0, :n, :] + wp_ref[1, :n, :] + v_ref[...])
        logits = logits * dis[:, None] + b_ref[...]
        m = jnp.max(logits, axis=-1, keepdims=True)
        ex = jnp.exp(logits - m)
        lse = jnp.log(jnp.sum(ex, axis=-1, keepdims=True)) + m
        o_ref[...] = logits - lse

    return pl.pallas_call(
        body,
        out_shape=jax.ShapeDtypeStruct(v.shape, jnp.float32),
    )(degp, wp, v, b2d)


def kernel(x, edge_index, W, b):
    n = x.shape[0]
    c_out = W.shape[1]
    e = edge_index.shape[1]

    n_pad = _cdiv(n + 1, _LANES) * _LANES
    nch = _cdiv(e, _LANES * _NW * 8) * _NW * 8
    nch_w = nch // _NW
    ep = nch * _LANES

    src = edge_index[0]
    dst = edge_index[1]
    pad = ep - e
    pad_i = jnp.arange(pad, dtype=src.dtype)
    src_p = jnp.concatenate(
        [src, pad_i % jnp.asarray(n, src.dtype)]).reshape(nch, _LANES)
    dst_p = jnp.concatenate(
        [dst, n + pad_i % jnp.asarray(n_pad - n, dst.dtype)]
    ).reshape(nch, _LANES)

    rows_w = n_pad // _NSUB
    zeros16 = jnp.zeros((rows_w, 16), jnp.float32)
    zerosf = jnp.zeros((rows_w, c_out), jnp.float32)
    ones16 = jnp.ones((_LANES, 16), jnp.float32)

    xw = _tc_matmul(x, W)
    degp = _sc_degree(dst_p, zeros16, ones16, n_pad, nch_w)
    z = _tc_scale_first(degp, xw, n)
    up = _sc_spmm(z, src_p, dst_p, zerosf, n_pad, nch_w)
    v = _tc_mid(degp, up, z, n)
    wp = _sc_spmm(v, src_p, dst_p, zerosf, n_pad, nch_w)
    return _tc_final(degp, wp, v, b.reshape(1, -1), n)
